# Initial kernel scaffold; baseline (speedup 1.0000x reference)
import jax
import jax.numpy as jnp
from jax.experimental import pallas as pl


def kernel(pc1, pc2, feat1, feat2, pos1_0_w, pos1_0_b, c11_0_w, c11_0_b, c12_0_w, c12_0_b, b1_0, pos1_1_w, pos1_1_b, c11_1_w, c11_1_b, c12_1_w, c12_1_b, b1_1, pos2_0_w, pos2_0_b, c21_0_w, c21_0_b, c22_0_w, c22_0_b, b2_0):
    raise NotImplementedError("write your pallas kernel here")


# speedup vs baseline: 20.4936x; 20.4936x over previous
import functools

import jax
import jax.numpy as jnp
from jax import lax
from jax.experimental import pallas as pl
from jax.experimental.pallas import tpu as pltpu
from jax.experimental.pallas import tpu_sc as plsc

B = 2
N = 4096
C = 64
K = 16
NDIR = 2
R = NDIR * B * N

RB = 256
PB = 512

NC, NS = 2, 16
NW = NC * NS


def _topk_body(xs_ref, xd_ref, out_ref):
    d_idx = pl.program_id(0)
    b_idx = pl.program_id(1)
    xs = xs_ref[0, 0]
    xd = xd_ref[0, 0]
    dot = lax.dot_general(xs, xd, (((0,), (0,)), ((), ())),
                          preferred_element_type=jnp.float32)
    ns = jnp.sum(xs * xs, axis=0)[:, None]
    nd = jnp.sum(xd * xd, axis=0)[None, :]
    d = ns + nd - 2.0 * dot

    iota = lax.broadcasted_iota(jnp.int32, (RB, N), 1)
    offset = (d_idx * B + b_idx) * N
    cols = []
    for _ in range(K):
        m = jnp.min(d, axis=1, keepdims=True)
        cand = jnp.where(d == m, iota, jnp.int32(N))
        amin = jnp.min(cand, axis=1, keepdims=True)
        cols.append(amin + offset)
        d = jnp.where(iota == amin, jnp.float32(jnp.inf), d)
    out_ref[0, 0] = jnp.concatenate(cols, axis=1)


def _topk(xcm):
    grid = (NDIR, B, N // RB)
    return pl.pallas_call(
        _topk_body,
        grid=grid,
        in_specs=[
            pl.BlockSpec((1, 1, 3, RB), lambda d, b, r: (d, b, 0, r)),
            pl.BlockSpec((1, 1, 3, N), lambda d, b, r: (1 - d, b, 0, 0)),
        ],
        out_specs=pl.BlockSpec((1, 1, RB, K), lambda d, b, r: (d, b, r, 0)),
        out_shape=jax.ShapeDtypeStruct((NDIR, B, N, K), jnp.int32),
    )(xcm, xcm)


def _prep_body(fa_ref, fb_ref, xa_ref, xb_ref,
               w1t_ref, w2t_ref, pwt_ref, cv1_ref, cv2_ref,
               a_ref, bse_ref):
    fa = fa_ref[0, 0]
    fb = fb_ref[0, 0]
    xa = xa_ref[0, 0]
    xb = xb_ref[0, 0]
    w1t = w1t_ref[...]
    w2t = w2t_ref[...]
    pwt = pwt_ref[...]
    a = (jnp.dot(fa, w2t, preferred_element_type=jnp.float32)
         + jnp.dot(xa, pwt, preferred_element_type=jnp.float32)
         + cv2_ref[...])
    bse = (jnp.dot(fb, w1t, preferred_element_type=jnp.float32)
           - jnp.dot(xb, pwt, preferred_element_type=jnp.float32)
           + cv1_ref[...])
    a_ref[0, 0] = a
    bse_ref[0, 0] = bse


def _prep(fpm, xpm, w1t, w2t, pwt, cv1, cv2):
    grid = (NDIR, B, N // PB)
    return pl.pallas_call(
        _prep_body,
        grid=grid,
        in_specs=[
            pl.BlockSpec((1, 1, PB, C), lambda d, b, p: (1 - d, b, p, 0)),
            pl.BlockSpec((1, 1, PB, C), lambda d, b, p: (d, b, p, 0)),
            pl.BlockSpec((1, 1, PB, 3), lambda d, b, p: (1 - d, b, p, 0)),
            pl.BlockSpec((1, 1, PB, 3), lambda d, b, p: (d, b, p, 0)),
            pl.BlockSpec((C, C), lambda d, b, p: (0, 0)),
            pl.BlockSpec((C, C), lambda d, b, p: (0, 0)),
            pl.BlockSpec((3, C), lambda d, b, p: (0, 0)),
            pl.BlockSpec((1, C), lambda d, b, p: (0, 0)),
            pl.BlockSpec((1, C), lambda d, b, p: (0, 0)),
        ],
        out_specs=[
            pl.BlockSpec((1, 1, PB, C), lambda d, b, p: (d, b, p, 0)),
            pl.BlockSpec((1, 1, PB, C), lambda d, b, p: (d, b, p, 0)),
        ],
        out_shape=[
            jax.ShapeDtypeStruct((NDIR, B, N, C), jnp.float32),
            jax.ShapeDtypeStruct((NDIR, B, N, C), jnp.float32),
        ],
    )(fpm, fpm, xpm, xpm, w1t, w2t, pwt, cv1, cv2)


@functools.lru_cache(maxsize=None)
def _make_gather_max(p_total):
    per_w = p_total // NW
    cp = 64
    nchunks = per_w // cp
    nstreams = (cp * K) // 128
    mesh = plsc.VectorSubcoreMesh(core_axis_name="c", subcore_axis_name="s",
                                  num_cores=NC, num_subcores=NS)

    @functools.partial(
        pl.kernel,
        out_type=jax.ShapeDtypeStruct((p_total, C), jnp.float32),
        mesh=mesh,
        compiler_params=pltpu.CompilerParams(use_tc_tiling_on_sc=False),
        scratch_types=[
            pltpu.VMEM((nstreams, 128), jnp.int32),
            pltpu.VMEM((cp * K, C), jnp.float32),
            pltpu.VMEM((cp, C), jnp.float32),
            pltpu.SemaphoreType.DMA,
        ],
    )
    def gather_max(table_hbm, idx_hbm, out_hbm, idx_v, rows_v, out_v, sem):
        wid = lax.axis_index("s") * NC + lax.axis_index("c")
        base_pt = wid * per_w

        def chunk_body(ci, carry):
            cbase = pl.multiple_of(base_pt + ci * cp, cp)
            irow = pl.multiple_of(cbase * K // 128, (cp * K) // 128)
            pltpu.sync_copy(idx_hbm.at[pl.ds(irow, nstreams)], idx_v)
            descs = []
            for j in range(nstreams):
                descs.append(pltpu.async_copy(
                    table_hbm.at[idx_v.at[j]],
                    rows_v.at[pl.ds(j * 128, 128)],
                    sem))
            for dsc in descs:
                dsc.wait()

            def pt_body(p, carry2):
                for q in range(C // 16):
                    sl = pl.ds(q * 16, 16)
                    acc = rows_v[p * K, sl]
                    for kk in range(1, K):
                        acc = jnp.maximum(acc, rows_v[p * K + kk, sl])
                    out_v[p, sl] = acc
                return carry2

            lax.fori_loop(0, cp, pt_body, 0)
            pltpu.sync_copy(out_v, out_hbm.at[pl.ds(cbase, cp)])
            return carry

        lax.fori_loop(0, nchunks, chunk_body, 0)

    return gather_max


def _gather_max(table, idx2d, p_total):
    return _make_gather_max(p_total)(table, idx2d)


def _leaky(x):
    return jnp.where(x >= 0, x, 0.1 * x)


def _post_body(bse_ref, m_ref, f_ref):
    f_ref[0, 0] = _leaky(bse_ref[0, 0] + m_ref[0, 0])


def _post(bse, m):
    grid = (NDIR, B, N // PB)
    spec = pl.BlockSpec((1, 1, PB, C), lambda d, b, p: (d, b, p, 0))
    return pl.pallas_call(
        _post_body,
        grid=grid,
        in_specs=[spec, spec],
        out_specs=spec,
        out_shape=jax.ShapeDtypeStruct((NDIR, B, N, C), jnp.float32),
    )(bse, m)


def _post_t_body(bse_ref, m_ref, f_ref, ft_ref):
    f = _leaky(bse_ref[0, 0] + m_ref[0, 0])
    f_ref[0, 0] = f
    ft_ref[0, 0] = f.T


def _post_t(bse, m):
    grid = (NDIR, B, N // PB)
    spec = pl.BlockSpec((1, 1, PB, C), lambda d, b, p: (d, b, p, 0))
    spec_t = pl.BlockSpec((1, 1, C, PB), lambda d, b, p: (d, b, 0, p))
    return pl.pallas_call(
        _post_t_body,
        grid=grid,
        in_specs=[spec, spec],
        out_specs=[spec, spec_t],
        out_shape=[
            jax.ShapeDtypeStruct((NDIR, B, N, C), jnp.float32),
            jax.ShapeDtypeStruct((NDIR, B, C, N), jnp.float32),
        ],
    )(bse, m)


def _post_t0_body(bse_ref, m_ref, ft_ref):
    ft_ref[0] = _leaky(bse_ref[0] + m_ref[0]).T


def _post_t0(bse, m):
    grid = (B, N // PB)
    spec = pl.BlockSpec((1, PB, C), lambda b, p: (b, p, 0))
    spec_t = pl.BlockSpec((1, C, PB), lambda b, p: (b, 0, p))
    return pl.pallas_call(
        _post_t0_body,
        grid=grid,
        in_specs=[spec, spec],
        out_specs=spec_t,
        out_shape=jax.ShapeDtypeStruct((B, C, N), jnp.float32),
    )(bse, m)


def kernel(pc1, pc2, feat1, feat2,
           pos1_0_w, pos1_0_b, c11_0_w, c11_0_b, c12_0_w, c12_0_b, b1_0,
           pos1_1_w, pos1_1_b, c11_1_w, c11_1_b, c12_1_w, c12_1_b, b1_1,
           pos2_0_w, pos2_0_b, c21_0_w, c21_0_b, c22_0_w, c22_0_b, b2_0):
    xcm = jnp.stack([pc1, pc2])
    xpm = xcm.transpose(0, 1, 3, 2)
    f0 = jnp.stack([feat1.transpose(0, 2, 1),
                    feat2.transpose(0, 2, 1)])

    idx = _topk(xcm)
    idx2d = idx.reshape(R * K // 128, 128)

    def layer_weights(pw, pb, w1, bb1, w2, bb2, bias):
        cv1 = (bb1 + pb + bias[0, :, 0, 0]).reshape(1, C)
        cv2 = bb2.reshape(1, C)
        return w1.T, w2.T, pw.T, cv1, cv2

    wl0 = layer_weights(pos1_0_w, pos1_0_b, c11_0_w, c11_0_b,
                        c12_0_w, c12_0_b, b1_0)
    wl1 = layer_weights(pos1_1_w, pos1_1_b, c11_1_w, c11_1_b,
                        c12_1_w, c12_1_b, b1_1)
    wl2 = layer_weights(pos2_0_w, pos2_0_b, c21_0_w, c21_0_b,
                        c22_0_w, c22_0_b, b2_0)

    a0, bse0 = _prep(f0, xpm, *wl0)
    m0 = _gather_max(a0.reshape(R, C), idx2d, R)
    f1 = _post(bse0, m0.reshape(NDIR, B, N, C))

    a1, bse1 = _prep(f1, xpm, *wl1)
    m1 = _gather_max(a1.reshape(R, C), idx2d, R)
    f2, f2t = _post_t(bse1, m1.reshape(NDIR, B, N, C))

    a2, bse2 = _prep(f2, xpm, *wl2)
    m2 = _gather_max(a2.reshape(R, C), idx2d[: R * K // 256], R // 2)
    final = _post_t0(bse2[0], m2.reshape(B, N, C))

    return (f2t[0], f2t[1], final)


# speedup vs baseline: 25.7728x; 1.2576x over previous
import functools

import jax
import jax.numpy as jnp
from jax import lax
from jax.experimental import pallas as pl
from jax.experimental.pallas import tpu as pltpu
from jax.experimental.pallas import tpu_sc as plsc

B = 2
N = 4096
C = 64
K = 16
NDIR = 2
R = NDIR * B * N

RB = 256
PB = 1024

NC, NS = 2, 16
NW = NC * NS


def _topk_body(xs_ref, xd_ref, out_ref):
    d_idx = pl.program_id(0)
    b_idx = pl.program_id(1)
    xs = xs_ref[0, 0]
    xd = xd_ref[0, 0]
    dot = lax.dot_general(xs, xd, (((0,), (0,)), ((), ())),
                          preferred_element_type=jnp.float32)
    ns = jnp.sum(xs * xs, axis=0)[:, None]
    nd = jnp.sum(xd * xd, axis=0)[None, :]
    d = ns + nd - 2.0 * dot

    fiota = lax.broadcasted_iota(jnp.int32, (RB, N), 1).astype(jnp.float32)
    offset = jnp.float32((d_idx * B + b_idx) * N)
    cols = []
    for _ in range(K):
        m = jnp.min(d, axis=1, keepdims=True)
        eq = d == m
        cand = jnp.where(eq, fiota, jnp.float32(1e9))
        amin = jnp.min(cand, axis=1, keepdims=True)
        cols.append(amin + offset)
        d = jnp.where(eq, jnp.float32(jnp.inf), d)
    out_ref[0, 0] = jnp.concatenate(cols, axis=1).astype(jnp.int32)


def _topk(xcm):
    grid = (NDIR, B, N // RB)
    return pl.pallas_call(
        _topk_body,
        grid=grid,
        in_specs=[
            pl.BlockSpec((1, 1, 3, RB), lambda d, b, r: (d, b, 0, r)),
            pl.BlockSpec((1, 1, 3, N), lambda d, b, r: (1 - d, b, 0, 0)),
        ],
        out_specs=pl.BlockSpec((1, 1, RB, K), lambda d, b, r: (d, b, r, 0)),
        out_shape=jax.ShapeDtypeStruct((NDIR, B, N, K), jnp.int32),
    )(xcm, xcm)


def _prep_body(fa_ref, fb_ref, xa_ref, xb_ref,
               w1t_ref, w2t_ref, pwt_ref, cv1_ref, cv2_ref,
               a_ref, bse_ref):
    fa = fa_ref[0, 0]
    fb = fb_ref[0, 0]
    xa = xa_ref[0, 0]
    xb = xb_ref[0, 0]
    w1t = w1t_ref[...]
    w2t = w2t_ref[...]
    pwt = pwt_ref[...]
    a = (jnp.dot(fa, w2t, preferred_element_type=jnp.float32)
         + jnp.dot(xa, pwt, preferred_element_type=jnp.float32)
         + cv2_ref[...])
    bse = (jnp.dot(fb, w1t, preferred_element_type=jnp.float32)
           - jnp.dot(xb, pwt, preferred_element_type=jnp.float32)
           + cv1_ref[...])
    a_ref[0, 0] = a
    bse_ref[0, 0] = bse


def _prep(fpm, xpm, w1t, w2t, pwt, cv1, cv2):
    grid = (NDIR, B, N // PB)
    return pl.pallas_call(
        _prep_body,
        grid=grid,
        in_specs=[
            pl.BlockSpec((1, 1, PB, C), lambda d, b, p: (1 - d, b, p, 0)),
            pl.BlockSpec((1, 1, PB, C), lambda d, b, p: (d, b, p, 0)),
            pl.BlockSpec((1, 1, PB, 3), lambda d, b, p: (1 - d, b, p, 0)),
            pl.BlockSpec((1, 1, PB, 3), lambda d, b, p: (d, b, p, 0)),
            pl.BlockSpec((C, C), lambda d, b, p: (0, 0)),
            pl.BlockSpec((C, C), lambda d, b, p: (0, 0)),
            pl.BlockSpec((3, C), lambda d, b, p: (0, 0)),
            pl.BlockSpec((1, C), lambda d, b, p: (0, 0)),
            pl.BlockSpec((1, C), lambda d, b, p: (0, 0)),
        ],
        out_specs=[
            pl.BlockSpec((1, 1, PB, C), lambda d, b, p: (d, b, p, 0)),
            pl.BlockSpec((1, 1, PB, C), lambda d, b, p: (d, b, p, 0)),
        ],
        out_shape=[
            jax.ShapeDtypeStruct((NDIR, B, N, C), jnp.float32),
            jax.ShapeDtypeStruct((NDIR, B, N, C), jnp.float32),
        ],
    )(fpm, fpm, xpm, xpm, w1t, w2t, pwt, cv1, cv2)


@functools.lru_cache(maxsize=None)
def _make_gather_max(p_total):
    per_w = p_total // NW
    cp = 64
    nchunks = per_w // cp
    nstreams = (cp * K) // 128
    mesh = plsc.VectorSubcoreMesh(core_axis_name="c", subcore_axis_name="s",
                                  num_cores=NC, num_subcores=NS)

    @functools.partial(
        pl.kernel,
        out_type=jax.ShapeDtypeStruct((p_total, C), jnp.float32),
        mesh=mesh,
        compiler_params=pltpu.CompilerParams(use_tc_tiling_on_sc=False),
        scratch_types=[
            pltpu.VMEM((nstreams, 128), jnp.int32),
            pltpu.VMEM((cp * K, C), jnp.float32),
            pltpu.VMEM((cp, C), jnp.float32),
            pltpu.SemaphoreType.DMA,
        ],
    )
    def gather_max(table_hbm, idx_hbm, out_hbm, idx_v, rows_v, out_v, sem):
        wid = lax.axis_index("s") * NC + lax.axis_index("c")
        base_pt = wid * per_w

        def chunk_body(ci, carry):
            cbase = pl.multiple_of(base_pt + ci * cp, cp)
            irow = pl.multiple_of(cbase * K // 128, (cp * K) // 128)
            pltpu.sync_copy(idx_hbm.at[pl.ds(irow, nstreams)], idx_v)
            descs = []
            for j in range(nstreams):
                descs.append(pltpu.async_copy(
                    table_hbm.at[idx_v.at[j]],
                    rows_v.at[pl.ds(j * 128, 128)],
                    sem))
            for dsc in descs:
                dsc.wait()

            def pt_body(p, carry2):
                for q in range(C // 16):
                    sl = pl.ds(q * 16, 16)
                    acc = rows_v[p * K, sl]
                    for kk in range(1, K):
                        acc = jnp.maximum(acc, rows_v[p * K + kk, sl])
                    out_v[p, sl] = acc
                return carry2

            lax.fori_loop(0, cp, pt_body, 0)
            pltpu.sync_copy(out_v, out_hbm.at[pl.ds(cbase, cp)])
            return carry

        lax.fori_loop(0, nchunks, chunk_body, 0)

    return gather_max


def _gather_max(table, idx2d, p_total):
    return _make_gather_max(p_total)(table, idx2d)


def _leaky(x):
    return jnp.where(x >= 0, x, 0.1 * x)


def _prep_fused_body(bsa_ref, ma_ref, bsb_ref, mb_ref, xa_ref, xb_ref,
                     w1t_ref, w2t_ref, pwt_ref, cv1_ref, cv2_ref,
                     a_ref, bse_ref):
    fa = _leaky(bsa_ref[0, 0] + ma_ref[0, 0])
    fb = _leaky(bsb_ref[0, 0] + mb_ref[0, 0])
    xa = xa_ref[0, 0]
    xb = xb_ref[0, 0]
    a = (jnp.dot(fa, w2t_ref[...], preferred_element_type=jnp.float32)
         + jnp.dot(xa, pwt_ref[...], preferred_element_type=jnp.float32)
         + cv2_ref[...])
    bse = (jnp.dot(fb, w1t_ref[...], preferred_element_type=jnp.float32)
           - jnp.dot(xb, pwt_ref[...], preferred_element_type=jnp.float32)
           + cv1_ref[...])
    a_ref[0, 0] = a
    bse_ref[0, 0] = bse


def _prep_fused(bse_prev, m_prev, xpm, w1t, w2t, pwt, cv1, cv2):
    grid = (NDIR, B, N // PB)
    spec_a = pl.BlockSpec((1, 1, PB, C), lambda d, b, p: (1 - d, b, p, 0))
    spec_b = pl.BlockSpec((1, 1, PB, C), lambda d, b, p: (d, b, p, 0))
    return pl.pallas_call(
        _prep_fused_body,
        grid=grid,
        in_specs=[
            spec_a, spec_a, spec_b, spec_b,
            pl.BlockSpec((1, 1, PB, 3), lambda d, b, p: (1 - d, b, p, 0)),
            pl.BlockSpec((1, 1, PB, 3), lambda d, b, p: (d, b, p, 0)),
            pl.BlockSpec((C, C), lambda d, b, p: (0, 0)),
            pl.BlockSpec((C, C), lambda d, b, p: (0, 0)),
            pl.BlockSpec((3, C), lambda d, b, p: (0, 0)),
            pl.BlockSpec((1, C), lambda d, b, p: (0, 0)),
            pl.BlockSpec((1, C), lambda d, b, p: (0, 0)),
        ],
        out_specs=[spec_b, spec_b],
        out_shape=[
            jax.ShapeDtypeStruct((NDIR, B, N, C), jnp.float32),
            jax.ShapeDtypeStruct((NDIR, B, N, C), jnp.float32),
        ],
    )(bse_prev, m_prev, bse_prev, m_prev, xpm, xpm,
      w1t, w2t, pwt, cv1, cv2)


def _post_t_body(bse_ref, m_ref, ft_ref):
    ft_ref[0, 0] = _leaky(bse_ref[0, 0] + m_ref[0, 0]).T


def _post_t(bse, m):
    grid = (NDIR, B, N // PB)
    spec = pl.BlockSpec((1, 1, PB, C), lambda d, b, p: (d, b, p, 0))
    spec_t = pl.BlockSpec((1, 1, C, PB), lambda d, b, p: (d, b, 0, p))
    return pl.pallas_call(
        _post_t_body,
        grid=grid,
        in_specs=[spec, spec],
        out_specs=spec_t,
        out_shape=jax.ShapeDtypeStruct((NDIR, B, C, N), jnp.float32),
    )(bse, m)


def _post_t0_body(bse_ref, m_ref, ft_ref):
    ft_ref[0] = _leaky(bse_ref[0] + m_ref[0]).T


def _post_t0(bse, m):
    grid = (B, N // PB)
    spec = pl.BlockSpec((1, PB, C), lambda b, p: (b, p, 0))
    spec_t = pl.BlockSpec((1, C, PB), lambda b, p: (b, 0, p))
    return pl.pallas_call(
        _post_t0_body,
        grid=grid,
        in_specs=[spec, spec],
        out_specs=spec_t,
        out_shape=jax.ShapeDtypeStruct((B, C, N), jnp.float32),
    )(bse, m)


def kernel(pc1, pc2, feat1, feat2,
           pos1_0_w, pos1_0_b, c11_0_w, c11_0_b, c12_0_w, c12_0_b, b1_0,
           pos1_1_w, pos1_1_b, c11_1_w, c11_1_b, c12_1_w, c12_1_b, b1_1,
           pos2_0_w, pos2_0_b, c21_0_w, c21_0_b, c22_0_w, c22_0_b, b2_0):
    xcm = jnp.stack([pc1, pc2])
    xpm = xcm.transpose(0, 1, 3, 2)
    f0 = jnp.stack([feat1.transpose(0, 2, 1),
                    feat2.transpose(0, 2, 1)])

    idx = _topk(xcm)
    idx2d = idx.reshape(R * K // 128, 128)

    def layer_weights(pw, pb, w1, bb1, w2, bb2, bias):
        cv1 = (bb1 + pb + bias[0, :, 0, 0]).reshape(1, C)
        cv2 = bb2.reshape(1, C)
        return w1.T, w2.T, pw.T, cv1, cv2

    wl0 = layer_weights(pos1_0_w, pos1_0_b, c11_0_w, c11_0_b,
                        c12_0_w, c12_0_b, b1_0)
    wl1 = layer_weights(pos1_1_w, pos1_1_b, c11_1_w, c11_1_b,
                        c12_1_w, c12_1_b, b1_1)
    wl2 = layer_weights(pos2_0_w, pos2_0_b, c21_0_w, c21_0_b,
                        c22_0_w, c22_0_b, b2_0)

    a0, bse0 = _prep(f0, xpm, *wl0)
    m0 = _gather_max(a0.reshape(R, C), idx2d, R)

    m0r = m0.reshape(NDIR, B, N, C)
    a1, bse1 = _prep_fused(bse0, m0r, xpm, *wl1)
    m1 = _gather_max(a1.reshape(R, C), idx2d, R)
    m1r = m1.reshape(NDIR, B, N, C)

    a2, bse2 = _prep_fused(bse1, m1r, xpm, *wl2)
    m2 = _gather_max(a2.reshape(R, C), idx2d[: R * K // 256], R // 2)
    final = _post_t0(bse2[0], m2.reshape(B, N, C))

    f2t = _post_t(bse1, m1r)

    return (f2t[0], f2t[1], final)


# speedup vs baseline: 26.1195x; 1.0135x over previous
import functools

import jax
import jax.numpy as jnp
from jax import lax
from jax.experimental import pallas as pl
from jax.experimental.pallas import tpu as pltpu
from jax.experimental.pallas import tpu_sc as plsc

B = 2
N = 4096
C = 64
K = 16
NDIR = 2
R = NDIR * B * N

RB = 256
PB = 1024

NC, NS = 2, 16
NW = NC * NS


def _topk_dir_body(d_idx, xs_ref, xd_ref, out_ref):
    b_idx = pl.program_id(0)
    xs = xs_ref[0, 0]
    xd = xd_ref[0, 0]
    dot = lax.dot_general(xs, xd, (((0,), (0,)), ((), ())),
                          preferred_element_type=jnp.float32)
    ns = jnp.sum(xs * xs, axis=0)[:, None]
    nd = jnp.sum(xd * xd, axis=0)[None, :]
    d = ns + nd - 2.0 * dot

    fiota = lax.broadcasted_iota(jnp.int32, (RB, N), 1).astype(jnp.float32)
    offset = (d_idx * B + b_idx) * jnp.float32(N)
    cols = []
    for _ in range(K):
        m = jnp.min(d, axis=1, keepdims=True)
        eq = d == m
        cand = jnp.where(eq, fiota, jnp.float32(1e9))
        amin = jnp.min(cand, axis=1, keepdims=True)
        cols.append(amin + offset)
        d = jnp.where(eq, jnp.float32(jnp.inf), d)
    out_ref[...] = jnp.concatenate(cols, axis=1).astype(jnp.int32)


def _topk_dir(xcm, d_idx):
    grid = (B, N // RB)
    out = pl.pallas_call(
        functools.partial(_topk_dir_body, d_idx),
        grid=grid,
        in_specs=[
            pl.BlockSpec((1, 1, 3, RB), lambda b, r: (d_idx, b, 0, r)),
            pl.BlockSpec((1, 1, 3, N), lambda b, r: (1 - d_idx, b, 0, 0)),
        ],
        out_specs=pl.BlockSpec((RB, K), lambda b, r: (b * (N // RB) + r, 0)),
        out_shape=jax.ShapeDtypeStruct((B * N, K), jnp.int32),
    )(xcm, xcm)
    return out.reshape(B * N * K // 128, 128)


def _flat_a(d, b, p):
    return ((1 - d) * B + b) * (N // PB) + p


def _flat_b(d, b, p):
    return (d * B + b) * (N // PB) + p


def _prep_tail(fa, fb, xa_ref, xb_ref, w1t_ref, w2t_ref, pwt_ref,
               cv1_ref, cv2_ref, a_ref, bse_ref):
    a = (jnp.dot(fa, w2t_ref[...], preferred_element_type=jnp.float32)
         + jnp.dot(xa_ref[0, 0], pwt_ref[...],
                   preferred_element_type=jnp.float32)
         + cv2_ref[...])
    bse = (jnp.dot(fb, w1t_ref[...], preferred_element_type=jnp.float32)
           - jnp.dot(xb_ref[0, 0], pwt_ref[...],
                     preferred_element_type=jnp.float32)
           + cv1_ref[...])
    a_ref[...] = a
    bse_ref[0, 0] = bse


_W_SPECS = [
    pl.BlockSpec((C, C), lambda d, b, p: (0, 0)),
    pl.BlockSpec((C, C), lambda d, b, p: (0, 0)),
    pl.BlockSpec((3, C), lambda d, b, p: (0, 0)),
    pl.BlockSpec((1, C), lambda d, b, p: (0, 0)),
    pl.BlockSpec((1, C), lambda d, b, p: (0, 0)),
]

_X_SPECS = [
    pl.BlockSpec((1, 1, PB, 3), lambda d, b, p: (1 - d, b, p, 0)),
    pl.BlockSpec((1, 1, PB, 3), lambda d, b, p: (d, b, p, 0)),
]

_OUT_SPECS = [
    pl.BlockSpec((PB, C), lambda d, b, p: (_flat_b(d, b, p), 0)),
    pl.BlockSpec((1, 1, PB, C), lambda d, b, p: (d, b, p, 0)),
]

_OUT_SHAPES = [
    jax.ShapeDtypeStruct((R, C), jnp.float32),
    jax.ShapeDtypeStruct((NDIR, B, N, C), jnp.float32),
]


def _prep_body(fa_ref, fb_ref, xa_ref, xb_ref,
               w1t_ref, w2t_ref, pwt_ref, cv1_ref, cv2_ref,
               a_ref, bse_ref):
    _prep_tail(fa_ref[0, 0], fb_ref[0, 0], xa_ref, xb_ref,
               w1t_ref, w2t_ref, pwt_ref, cv1_ref, cv2_ref, a_ref, bse_ref)


def _prep(fpm, xpm, w1t, w2t, pwt, cv1, cv2):
    grid = (NDIR, B, N // PB)
    return pl.pallas_call(
        _prep_body,
        grid=grid,
        in_specs=[
            pl.BlockSpec((1, 1, PB, C), lambda d, b, p: (1 - d, b, p, 0)),
            pl.BlockSpec((1, 1, PB, C), lambda d, b, p: (d, b, p, 0)),
            *_X_SPECS,
            *_W_SPECS,
        ],
        out_specs=_OUT_SPECS,
        out_shape=_OUT_SHAPES,
    )(fpm, fpm, xpm, xpm, w1t, w2t, pwt, cv1, cv2)


def _leaky(x):
    return jnp.where(x >= 0, x, 0.1 * x)


def _prep_fused_body(bsa_ref, ma_ref, bsb_ref, mb_ref, xa_ref, xb_ref,
                     w1t_ref, w2t_ref, pwt_ref, cv1_ref, cv2_ref,
                     a_ref, bse_ref):
    fa = _leaky(bsa_ref[0, 0] + ma_ref[...])
    fb = _leaky(bsb_ref[0, 0] + mb_ref[...])
    _prep_tail(fa, fb, xa_ref, xb_ref,
               w1t_ref, w2t_ref, pwt_ref, cv1_ref, cv2_ref, a_ref, bse_ref)


def _prep_fused(bse_prev, m_prev, xpm, w1t, w2t, pwt, cv1, cv2):
    grid = (NDIR, B, N // PB)
    return pl.pallas_call(
        _prep_fused_body,
        grid=grid,
        in_specs=[
            pl.BlockSpec((1, 1, PB, C), lambda d, b, p: (1 - d, b, p, 0)),
            pl.BlockSpec((PB, C), lambda d, b, p: (_flat_a(d, b, p), 0)),
            pl.BlockSpec((1, 1, PB, C), lambda d, b, p: (d, b, p, 0)),
            pl.BlockSpec((PB, C), lambda d, b, p: (_flat_b(d, b, p), 0)),
            *_X_SPECS,
            *_W_SPECS,
        ],
        out_specs=_OUT_SPECS,
        out_shape=_OUT_SHAPES,
    )(bse_prev, m_prev, bse_prev, m_prev, xpm, xpm,
      w1t, w2t, pwt, cv1, cv2)


@functools.lru_cache(maxsize=None)
def _make_gather_max(p_total):
    per_w = p_total // NW
    cp = 64
    nchunks = per_w // cp
    nstreams = (cp * K) // 128
    mesh = plsc.VectorSubcoreMesh(core_axis_name="c", subcore_axis_name="s",
                                  num_cores=NC, num_subcores=NS)

    @functools.partial(
        pl.kernel,
        out_type=jax.ShapeDtypeStruct((p_total, C), jnp.float32),
        mesh=mesh,
        compiler_params=pltpu.CompilerParams(use_tc_tiling_on_sc=False),
        scratch_types=[
            pltpu.VMEM((nstreams, 128), jnp.int32),
            pltpu.VMEM((cp * K, C), jnp.float32),
            pltpu.VMEM((cp, C), jnp.float32),
            pltpu.SemaphoreType.DMA,
        ],
    )
    def gather_max(table_hbm, idx_hbm, out_hbm, idx_v, rows_v, out_v, sem):
        wid = lax.axis_index("s") * NC + lax.axis_index("c")
        base_pt = wid * per_w

        def chunk_body(ci, carry):
            cbase = pl.multiple_of(base_pt + ci * cp, cp)
            irow = pl.multiple_of(cbase * K // 128, (cp * K) // 128)
            pltpu.sync_copy(idx_hbm.at[pl.ds(irow, nstreams)], idx_v)
            descs = []
            for j in range(nstreams):
                descs.append(pltpu.async_copy(
                    table_hbm.at[idx_v.at[j]],
                    rows_v.at[pl.ds(j * 128, 128)],
                    sem))
            for dsc in descs:
                dsc.wait()

            def pt_body(p, carry2):
                for q in range(C // 16):
                    sl = pl.ds(q * 16, 16)
                    acc = rows_v[p * K, sl]
                    for kk in range(1, K):
                        acc = jnp.maximum(acc, rows_v[p * K + kk, sl])
                    out_v[p, sl] = acc
                return carry2

            lax.fori_loop(0, cp, pt_body, 0)
            pltpu.sync_copy(out_v, out_hbm.at[pl.ds(cbase, cp)])
            return carry

        lax.fori_loop(0, nchunks, chunk_body, 0)

    return gather_max


def _gather_max(table, idx2d, p_total):
    return _make_gather_max(p_total)(table, idx2d)


def _post_t2_body(bs0_ref, m0_ref, bs1_ref, m1_ref, f1t_ref, f2t_ref):
    f1t_ref[0] = _leaky(bs0_ref[0, 0] + m0_ref[...]).T
    f2t_ref[0] = _leaky(bs1_ref[0, 0] + m1_ref[...]).T


def _post_t2(bse, m):
    grid = (B, N // PB)
    spec_t = pl.BlockSpec((1, C, PB), lambda b, p: (b, 0, p))
    out_sh = jax.ShapeDtypeStruct((B, C, N), jnp.float32)
    return pl.pallas_call(
        _post_t2_body,
        grid=grid,
        in_specs=[
            pl.BlockSpec((1, 1, PB, C), lambda b, p: (0, b, p, 0)),
            pl.BlockSpec((PB, C), lambda b, p: (b * (N // PB) + p, 0)),
            pl.BlockSpec((1, 1, PB, C), lambda b, p: (1, b, p, 0)),
            pl.BlockSpec((PB, C), lambda b, p: ((B + b) * (N // PB) + p, 0)),
        ],
        out_specs=[spec_t, spec_t],
        out_shape=[out_sh, out_sh],
    )(bse, m, bse, m)


def _post_t0_body(bse_ref, m_ref, ft_ref):
    ft_ref[0] = _leaky(bse_ref[0, 0] + m_ref[...]).T


def _post_t0(bse, m):
    grid = (B, N // PB)
    return pl.pallas_call(
        _post_t0_body,
        grid=grid,
        in_specs=[
            pl.BlockSpec((1, 1, PB, C), lambda b, p: (0, b, p, 0)),
            pl.BlockSpec((PB, C), lambda b, p: (b * (N // PB) + p, 0)),
        ],
        out_specs=pl.BlockSpec((1, C, PB), lambda b, p: (b, 0, p)),
        out_shape=jax.ShapeDtypeStruct((B, C, N), jnp.float32),
    )(bse, m)


def kernel(pc1, pc2, feat1, feat2,
           pos1_0_w, pos1_0_b, c11_0_w, c11_0_b, c12_0_w, c12_0_b, b1_0,
           pos1_1_w, pos1_1_b, c11_1_w, c11_1_b, c12_1_w, c12_1_b, b1_1,
           pos2_0_w, pos2_0_b, c21_0_w, c21_0_b, c22_0_w, c22_0_b, b2_0):
    xcm = jnp.stack([pc1, pc2])
    xpm = xcm.transpose(0, 1, 3, 2)
    f0 = jnp.stack([feat1.transpose(0, 2, 1),
                    feat2.transpose(0, 2, 1)])

    def layer_weights(pw, pb, w1, bb1, w2, bb2, bias):
        cv1 = (bb1 + pb + bias[0, :, 0, 0]).reshape(1, C)
        cv2 = bb2.reshape(1, C)
        return w1.T, w2.T, pw.T, cv1, cv2

    wl0 = layer_weights(pos1_0_w, pos1_0_b, c11_0_w, c11_0_b,
                        c12_0_w, c12_0_b, b1_0)
    wl1 = layer_weights(pos1_1_w, pos1_1_b, c11_1_w, c11_1_b,
                        c12_1_w, c12_1_b, b1_1)
    wl2 = layer_weights(pos2_0_w, pos2_0_b, c21_0_w, c21_0_b,
                        c22_0_w, c22_0_b, b2_0)

    idx_d0 = _topk_dir(xcm, 0)
    a0, bse0 = _prep(f0, xpm, *wl0)
    m0_d0 = _gather_max(a0, idx_d0, R // 2)
    idx_d1 = _topk_dir(xcm, 1)
    m0_d1 = _gather_max(a0, idx_d1, R // 2)
    m0 = jnp.concatenate([m0_d0, m0_d1], axis=0)
    idx_all = jnp.concatenate([idx_d0, idx_d1], axis=0)

    a1, bse1 = _prep_fused(bse0, m0, xpm, *wl1)
    m1 = _gather_max(a1, idx_all, R)

    a2, bse2 = _prep_fused(bse1, m1, xpm, *wl2)
    m2 = _gather_max(a2, idx_d0, R // 2)

    f1t, f2t = _post_t2(bse1, m1)
    final = _post_t0(bse2, m2)

    return (f1t, f2t, final)


# speedup vs baseline: 27.1991x; 1.0413x over previous
import functools

import jax
import jax.numpy as jnp
from jax import lax
from jax.experimental import pallas as pl
from jax.experimental.pallas import tpu as pltpu
from jax.experimental.pallas import tpu_sc as plsc

B = 2
N = 4096
C = 64
K = 16
NDIR = 2
R = NDIR * B * N

RB = 256
PB = 1024

NC, NS = 2, 16
NW = NC * NS


def _topk_dir_body(d_idx, xs_ref, xd_ref, out_ref):
    b_idx = pl.program_id(0)
    xs = xs_ref[0, 0]
    xd = xd_ref[0, 0]
    dot = lax.dot_general(xs, xd, (((0,), (0,)), ((), ())),
                          preferred_element_type=jnp.float32)
    ns = jnp.sum(xs * xs, axis=0)[:, None]
    nd = jnp.sum(xd * xd, axis=0)[None, :]
    d = ns + nd - 2.0 * dot

    fiota = lax.broadcasted_iota(jnp.int32, (RB, N), 1).astype(jnp.float32)
    offset = (d_idx * B + b_idx) * jnp.float32(N)
    cols = []
    for _ in range(K):
        m = jnp.min(d, axis=1, keepdims=True)
        eq = d == m
        cand = jnp.where(eq, fiota, jnp.float32(1e9))
        amin = jnp.min(cand, axis=1, keepdims=True)
        cols.append(amin + offset)
        d = jnp.where(eq, jnp.float32(jnp.inf), d)
    out_ref[...] = jnp.concatenate(cols, axis=1).astype(jnp.int32)


def _topk_dir(xcm, d_idx):
    grid = (B, N // RB)
    out = pl.pallas_call(
        functools.partial(_topk_dir_body, d_idx),
        grid=grid,
        in_specs=[
            pl.BlockSpec((1, 1, 3, RB), lambda b, r: (d_idx, b, 0, r)),
            pl.BlockSpec((1, 1, 3, N), lambda b, r: (1 - d_idx, b, 0, 0)),
        ],
        out_specs=pl.BlockSpec((RB, K), lambda b, r: (b * (N // RB) + r, 0)),
        out_shape=jax.ShapeDtypeStruct((B * N, K), jnp.int32),
    )(xcm, xcm)
    return out.reshape(B * N * K // 128, 128)


def _flat_a(d, b, p):
    return ((1 - d) * B + b) * (N // PB) + p


def _flat_b(d, b, p):
    return (d * B + b) * (N // PB) + p


def _prep_tail(fa, fb, xa_ref, xb_ref, w1t_ref, w2t_ref, pwt_ref,
               cv1_ref, cv2_ref, a_ref, bse_ref):
    a = (jnp.dot(fa, w2t_ref[...], preferred_element_type=jnp.float32)
         + jnp.dot(xa_ref[0, 0], pwt_ref[...],
                   preferred_element_type=jnp.float32)
         + cv2_ref[...])
    bse = (jnp.dot(fb, w1t_ref[...], preferred_element_type=jnp.float32)
           - jnp.dot(xb_ref[0, 0], pwt_ref[...],
                     preferred_element_type=jnp.float32)
           + cv1_ref[...])
    a_ref[...] = a
    bse_ref[0, 0] = bse


_W_SPECS = [
    pl.BlockSpec((C, C), lambda d, b, p: (0, 0)),
    pl.BlockSpec((C, C), lambda d, b, p: (0, 0)),
    pl.BlockSpec((3, C), lambda d, b, p: (0, 0)),
    pl.BlockSpec((1, C), lambda d, b, p: (0, 0)),
    pl.BlockSpec((1, C), lambda d, b, p: (0, 0)),
]

_X_SPECS = [
    pl.BlockSpec((1, 1, PB, 3), lambda d, b, p: (1 - d, b, p, 0)),
    pl.BlockSpec((1, 1, PB, 3), lambda d, b, p: (d, b, p, 0)),
]

_OUT_SPECS = [
    pl.BlockSpec((PB, C), lambda d, b, p: (_flat_b(d, b, p), 0)),
    pl.BlockSpec((1, 1, PB, C), lambda d, b, p: (d, b, p, 0)),
]

_OUT_SHAPES = [
    jax.ShapeDtypeStruct((R, C), jnp.float32),
    jax.ShapeDtypeStruct((NDIR, B, N, C), jnp.float32),
]


def _prep_body(fa_ref, fb_ref, xa_ref, xb_ref,
               w1t_ref, w2t_ref, pwt_ref, cv1_ref, cv2_ref,
               a_ref, bse_ref):
    _prep_tail(fa_ref[0, 0], fb_ref[0, 0], xa_ref, xb_ref,
               w1t_ref, w2t_ref, pwt_ref, cv1_ref, cv2_ref, a_ref, bse_ref)


def _prep(fpm, xpm, w1t, w2t, pwt, cv1, cv2):
    grid = (NDIR, B, N // PB)
    return pl.pallas_call(
        _prep_body,
        grid=grid,
        in_specs=[
            pl.BlockSpec((1, 1, PB, C), lambda d, b, p: (1 - d, b, p, 0)),
            pl.BlockSpec((1, 1, PB, C), lambda d, b, p: (d, b, p, 0)),
            *_X_SPECS,
            *_W_SPECS,
        ],
        out_specs=_OUT_SPECS,
        out_shape=_OUT_SHAPES,
    )(fpm, fpm, xpm, xpm, w1t, w2t, pwt, cv1, cv2)


def _leaky(x):
    return jnp.where(x >= 0, x, 0.1 * x)


def _prep_fused_body(bsa_ref, ma_ref, bsb_ref, mb_ref, xa_ref, xb_ref,
                     w1t_ref, w2t_ref, pwt_ref, cv1_ref, cv2_ref,
                     a_ref, bse_ref):
    fa = _leaky(bsa_ref[0, 0] + ma_ref[...])
    fb = _leaky(bsb_ref[0, 0] + mb_ref[...])
    _prep_tail(fa, fb, xa_ref, xb_ref,
               w1t_ref, w2t_ref, pwt_ref, cv1_ref, cv2_ref, a_ref, bse_ref)


def _prep_fused(bse_prev, m_prev, xpm, w1t, w2t, pwt, cv1, cv2):
    grid = (NDIR, B, N // PB)
    return pl.pallas_call(
        _prep_fused_body,
        grid=grid,
        in_specs=[
            pl.BlockSpec((1, 1, PB, C), lambda d, b, p: (1 - d, b, p, 0)),
            pl.BlockSpec((PB, C), lambda d, b, p: (_flat_a(d, b, p), 0)),
            pl.BlockSpec((1, 1, PB, C), lambda d, b, p: (d, b, p, 0)),
            pl.BlockSpec((PB, C), lambda d, b, p: (_flat_b(d, b, p), 0)),
            *_X_SPECS,
            *_W_SPECS,
        ],
        out_specs=_OUT_SPECS,
        out_shape=_OUT_SHAPES,
    )(bse_prev, m_prev, bse_prev, m_prev, xpm, xpm,
      w1t, w2t, pwt, cv1, cv2)


@functools.lru_cache(maxsize=None)
def _make_gather_max(p_total):
    per_w = p_total // NW
    cp = 32
    nchunks = per_w // cp
    nstreams = (cp * K) // 128
    nrows_i = per_w * K // 128
    mesh = plsc.VectorSubcoreMesh(core_axis_name="c", subcore_axis_name="s",
                                  num_cores=NC, num_subcores=NS)

    @functools.partial(
        pl.kernel,
        out_type=jax.ShapeDtypeStruct((p_total, C), jnp.float32),
        mesh=mesh,
        compiler_params=pltpu.CompilerParams(use_tc_tiling_on_sc=False),
        scratch_types=[
            pltpu.VMEM((nrows_i, 128), jnp.int32),
            pltpu.VMEM((cp * K, C), jnp.float32),
            pltpu.VMEM((cp * K, C), jnp.float32),
            pltpu.VMEM((cp, C), jnp.float32),
            pltpu.SemaphoreType.DMA,
            pltpu.SemaphoreType.DMA,
        ],
    )
    def gather_max(table_hbm, idx_hbm, out_hbm,
                   idx_v, rows0, rows1, out_v, sem0, sem1):
        wid = lax.axis_index("s") * NC + lax.axis_index("c")
        base_pt = wid * per_w
        irow = pl.multiple_of(base_pt * K // 128, nrows_i)
        pltpu.sync_copy(idx_hbm.at[pl.ds(irow, nrows_i)], idx_v)

        def fire(ci, buf, sem):
            for j in range(nstreams):
                pltpu.async_copy(table_hbm.at[idx_v.at[ci * nstreams + j]],
                                 buf.at[pl.ds(j * 128, 128)], sem)

        def drain(buf, sem):
            pltpu.make_async_copy(table_hbm.at[pl.ds(0, cp * K)],
                                  buf, sem).wait()

        def compute(ci, buf):
            def pt_body(p, carry):
                for q in range(C // 16):
                    sl = pl.ds(q * 16, 16)
                    acc = buf[p * K, sl]
                    for kk in range(1, K):
                        acc = jnp.maximum(acc, buf[p * K + kk, sl])
                    out_v[p, sl] = acc
                return carry

            lax.fori_loop(0, cp, pt_body, 0)
            cbase = pl.multiple_of(base_pt + ci * cp, cp)
            pltpu.sync_copy(out_v, out_hbm.at[pl.ds(cbase, cp)])

        fire(0, rows0, sem0)

        def pair_body(g, carry):
            c0 = 2 * g
            fire(c0 + 1, rows1, sem1)
            drain(rows0, sem0)
            compute(c0, rows0)

            @pl.when(c0 + 2 < nchunks)
            def _():
                fire(c0 + 2, rows0, sem0)

            drain(rows1, sem1)
            compute(c0 + 1, rows1)
            return carry

        lax.fori_loop(0, nchunks // 2, pair_body, 0)

    return gather_max


def _gather_max(table, idx2d, p_total):
    return _make_gather_max(p_total)(table, idx2d)


def _post_t2_body(bs0_ref, m0_ref, bs1_ref, m1_ref, f1t_ref, f2t_ref):
    f1t_ref[0] = _leaky(bs0_ref[0, 0] + m0_ref[...]).T
    f2t_ref[0] = _leaky(bs1_ref[0, 0] + m1_ref[...]).T


def _post_t2(bse, m):
    grid = (B, N // PB)
    spec_t = pl.BlockSpec((1, C, PB), lambda b, p: (b, 0, p))
    out_sh = jax.ShapeDtypeStruct((B, C, N), jnp.float32)
    return pl.pallas_call(
        _post_t2_body,
        grid=grid,
        in_specs=[
            pl.BlockSpec((1, 1, PB, C), lambda b, p: (0, b, p, 0)),
            pl.BlockSpec((PB, C), lambda b, p: (b * (N // PB) + p, 0)),
            pl.BlockSpec((1, 1, PB, C), lambda b, p: (1, b, p, 0)),
            pl.BlockSpec((PB, C), lambda b, p: ((B + b) * (N // PB) + p, 0)),
        ],
        out_specs=[spec_t, spec_t],
        out_shape=[out_sh, out_sh],
    )(bse, m, bse, m)


def _post_t0_body(bse_ref, m_ref, ft_ref):
    ft_ref[0] = _leaky(bse_ref[0, 0] + m_ref[...]).T


def _post_t0(bse, m):
    grid = (B, N // PB)
    return pl.pallas_call(
        _post_t0_body,
        grid=grid,
        in_specs=[
            pl.BlockSpec((1, 1, PB, C), lambda b, p: (0, b, p, 0)),
            pl.BlockSpec((PB, C), lambda b, p: (b * (N // PB) + p, 0)),
        ],
        out_specs=pl.BlockSpec((1, C, PB), lambda b, p: (b, 0, p)),
        out_shape=jax.ShapeDtypeStruct((B, C, N), jnp.float32),
    )(bse, m)


def kernel(pc1, pc2, feat1, feat2,
           pos1_0_w, pos1_0_b, c11_0_w, c11_0_b, c12_0_w, c12_0_b, b1_0,
           pos1_1_w, pos1_1_b, c11_1_w, c11_1_b, c12_1_w, c12_1_b, b1_1,
           pos2_0_w, pos2_0_b, c21_0_w, c21_0_b, c22_0_w, c22_0_b, b2_0):
    xcm = jnp.stack([pc1, pc2])
    xpm = xcm.transpose(0, 1, 3, 2)
    f0 = jnp.stack([feat1.transpose(0, 2, 1),
                    feat2.transpose(0, 2, 1)])

    def layer_weights(pw, pb, w1, bb1, w2, bb2, bias):
        cv1 = (bb1 + pb + bias[0, :, 0, 0]).reshape(1, C)
        cv2 = bb2.reshape(1, C)
        return w1.T, w2.T, pw.T, cv1, cv2

    wl0 = layer_weights(pos1_0_w, pos1_0_b, c11_0_w, c11_0_b,
                        c12_0_w, c12_0_b, b1_0)
    wl1 = layer_weights(pos1_1_w, pos1_1_b, c11_1_w, c11_1_b,
                        c12_1_w, c12_1_b, b1_1)
    wl2 = layer_weights(pos2_0_w, pos2_0_b, c21_0_w, c21_0_b,
                        c22_0_w, c22_0_b, b2_0)

    idx_d0 = _topk_dir(xcm, 0)
    a0, bse0 = _prep(f0, xpm, *wl0)
    m0_d0 = _gather_max(a0, idx_d0, R // 2)
    idx_d1 = _topk_dir(xcm, 1)
    m0_d1 = _gather_max(a0, idx_d1, R // 2)
    m0 = jnp.concatenate([m0_d0, m0_d1], axis=0)
    idx_all = jnp.concatenate([idx_d0, idx_d1], axis=0)

    a1, bse1 = _prep_fused(bse0, m0, xpm, *wl1)
    m1 = _gather_max(a1, idx_all, R)

    a2, bse2 = _prep_fused(bse1, m1, xpm, *wl2)
    m2 = _gather_max(a2, idx_d0, R // 2)

    f1t, f2t = _post_t2(bse1, m1)
    final = _post_t0(bse2, m2)

    return (f1t, f2t, final)


# speedup vs baseline: 27.4230x; 1.0082x over previous
import functools

import jax
import jax.numpy as jnp
from jax import lax
from jax.experimental import pallas as pl
from jax.experimental.pallas import tpu as pltpu
from jax.experimental.pallas import tpu_sc as plsc

B = 2
N = 4096
C = 64
K = 16
NDIR = 2
R = NDIR * B * N

RB = 256
PB = 1024

NC, NS = 2, 16
NW = NC * NS


def _topk_dir_body(d_idx, xs_ref, xd_ref, out_ref):
    b_idx = pl.program_id(0)
    xs = xs_ref[0, 0]
    xd = xd_ref[0, 0]
    dot = lax.dot_general(xs, xd, (((0,), (0,)), ((), ())),
                          preferred_element_type=jnp.float32)
    ns = jnp.sum(xs * xs, axis=0)[:, None]
    nd = jnp.sum(xd * xd, axis=0)[None, :]
    d = ns + nd - 2.0 * dot

    fiota = lax.broadcasted_iota(jnp.int32, (RB, N), 1).astype(jnp.float32)
    offset = (d_idx * B + b_idx) * jnp.float32(N)
    cols = []
    for _ in range(K):
        m = jnp.min(d, axis=1, keepdims=True)
        eq = d == m
        cand = jnp.where(eq, fiota, jnp.float32(1e9))
        amin = jnp.min(cand, axis=1, keepdims=True)
        cols.append(amin + offset)
        d = jnp.where(eq, jnp.float32(jnp.inf), d)
    out_ref[...] = jnp.concatenate(cols, axis=1).astype(jnp.int32)


def _topk_dir(xcm, d_idx):
    grid = (B, N // RB)
    out = pl.pallas_call(
        functools.partial(_topk_dir_body, d_idx),
        grid=grid,
        in_specs=[
            pl.BlockSpec((1, 1, 3, RB), lambda b, r: (d_idx, b, 0, r)),
            pl.BlockSpec((1, 1, 3, N), lambda b, r: (1 - d_idx, b, 0, 0)),
        ],
        out_specs=pl.BlockSpec((RB, K), lambda b, r: (b * (N // RB) + r, 0)),
        out_shape=jax.ShapeDtypeStruct((B * N, K), jnp.int32),
    )(xcm, xcm)
    return out.reshape(B * N * K // 128, 128)


def _flat_a(d, b, p):
    return ((1 - d) * B + b) * (N // PB) + p


def _flat_b(d, b, p):
    return (d * B + b) * (N // PB) + p


def _prep_tail(fa, fb, xa_ref, xb_ref, w1t_ref, w2t_ref, pwt_ref,
               cv1_ref, cv2_ref, a_ref, bse_ref):
    a = (jnp.dot(fa, w2t_ref[...], preferred_element_type=jnp.float32)
         + jnp.dot(xa_ref[0, 0], pwt_ref[...],
                   preferred_element_type=jnp.float32)
         + cv2_ref[...])
    bse = (jnp.dot(fb, w1t_ref[...], preferred_element_type=jnp.float32)
           - jnp.dot(xb_ref[0, 0], pwt_ref[...],
                     preferred_element_type=jnp.float32)
           + cv1_ref[...])
    a_ref[...] = a
    bse_ref[0, 0] = bse


_W_SPECS = [
    pl.BlockSpec((C, C), lambda d, b, p: (0, 0)),
    pl.BlockSpec((C, C), lambda d, b, p: (0, 0)),
    pl.BlockSpec((3, C), lambda d, b, p: (0, 0)),
    pl.BlockSpec((1, C), lambda d, b, p: (0, 0)),
    pl.BlockSpec((1, C), lambda d, b, p: (0, 0)),
]

_X_SPECS = [
    pl.BlockSpec((1, 1, PB, 3), lambda d, b, p: (1 - d, b, p, 0)),
    pl.BlockSpec((1, 1, PB, 3), lambda d, b, p: (d, b, p, 0)),
]

_OUT_SPECS = [
    pl.BlockSpec((PB, C), lambda d, b, p: (_flat_b(d, b, p), 0)),
    pl.BlockSpec((1, 1, PB, C), lambda d, b, p: (d, b, p, 0)),
]

_OUT_SHAPES = [
    jax.ShapeDtypeStruct((R, C), jnp.float32),
    jax.ShapeDtypeStruct((NDIR, B, N, C), jnp.float32),
]


def _prep_body(fa_ref, fb_ref, xa_ref, xb_ref,
               w1t_ref, w2t_ref, pwt_ref, cv1_ref, cv2_ref,
               a_ref, bse_ref):
    _prep_tail(fa_ref[0, 0], fb_ref[0, 0], xa_ref, xb_ref,
               w1t_ref, w2t_ref, pwt_ref, cv1_ref, cv2_ref, a_ref, bse_ref)


def _prep(fpm, xpm, w1t, w2t, pwt, cv1, cv2):
    grid = (NDIR, B, N // PB)
    return pl.pallas_call(
        _prep_body,
        grid=grid,
        in_specs=[
            pl.BlockSpec((1, 1, PB, C), lambda d, b, p: (1 - d, b, p, 0)),
            pl.BlockSpec((1, 1, PB, C), lambda d, b, p: (d, b, p, 0)),
            *_X_SPECS,
            *_W_SPECS,
        ],
        out_specs=_OUT_SPECS,
        out_shape=_OUT_SHAPES,
    )(fpm, fpm, xpm, xpm, w1t, w2t, pwt, cv1, cv2)


def _leaky(x):
    return jnp.where(x >= 0, x, 0.1 * x)


def _prep_fused_body(bsa_ref, bsb_ref, mlo_ref, mhi_ref, xa_ref, xb_ref,
                     w1t_ref, w2t_ref, pwt_ref, cv1_ref, cv2_ref,
                     a_ref, bse_ref):
    d0 = pl.program_id(0) == 0
    mlo = mlo_ref[...]
    mhi = mhi_ref[...]
    ma = jnp.where(d0, mhi, mlo)
    mb = jnp.where(d0, mlo, mhi)
    fa = _leaky(bsa_ref[0, 0] + ma)
    fb = _leaky(bsb_ref[0, 0] + mb)
    _prep_tail(fa, fb, xa_ref, xb_ref,
               w1t_ref, w2t_ref, pwt_ref, cv1_ref, cv2_ref, a_ref, bse_ref)


def _prep_fused(bse_prev, m_lo, m_hi, xpm, w1t, w2t, pwt, cv1, cv2):
    grid = (NDIR, B, N // PB)
    spec_m = pl.BlockSpec((PB, C), lambda d, b, p: (b * (N // PB) + p, 0))
    return pl.pallas_call(
        _prep_fused_body,
        grid=grid,
        in_specs=[
            pl.BlockSpec((1, 1, PB, C), lambda d, b, p: (1 - d, b, p, 0)),
            pl.BlockSpec((1, 1, PB, C), lambda d, b, p: (d, b, p, 0)),
            spec_m, spec_m,
            *_X_SPECS,
            *_W_SPECS,
        ],
        out_specs=_OUT_SPECS,
        out_shape=_OUT_SHAPES,
    )(bse_prev, bse_prev, m_lo, m_hi, xpm, xpm,
      w1t, w2t, pwt, cv1, cv2)


@functools.lru_cache(maxsize=None)
def _make_gather_max(p_total):
    per_w = p_total // NW
    cp = 32
    nchunks = per_w // cp
    nstreams = (cp * K) // 128
    nrows_i = per_w * K // 128
    mesh = plsc.VectorSubcoreMesh(core_axis_name="c", subcore_axis_name="s",
                                  num_cores=NC, num_subcores=NS)

    @functools.partial(
        pl.kernel,
        out_type=jax.ShapeDtypeStruct((p_total, C), jnp.float32),
        mesh=mesh,
        compiler_params=pltpu.CompilerParams(use_tc_tiling_on_sc=False),
        scratch_types=[
            pltpu.VMEM((nrows_i, 128), jnp.int32),
            pltpu.VMEM((cp * K, C), jnp.float32),
            pltpu.VMEM((cp * K, C), jnp.float32),
            pltpu.VMEM((cp, C), jnp.float32),
            pltpu.SemaphoreType.DMA,
            pltpu.SemaphoreType.DMA,
        ],
    )
    def gather_max(table_hbm, idx_hbm, out_hbm,
                   idx_v, rows0, rows1, out_v, sem0, sem1):
        wid = lax.axis_index("s") * NC + lax.axis_index("c")
        base_pt = wid * per_w
        irow = pl.multiple_of(base_pt * K // 128, nrows_i)
        pltpu.sync_copy(idx_hbm.at[pl.ds(irow, nrows_i)], idx_v)

        def fire(ci, buf, sem):
            for j in range(nstreams):
                pltpu.async_copy(table_hbm.at[idx_v.at[ci * nstreams + j]],
                                 buf.at[pl.ds(j * 128, 128)], sem)

        def drain(buf, sem):
            pltpu.make_async_copy(table_hbm.at[pl.ds(0, cp * K)],
                                  buf, sem).wait()

        def compute(ci, buf):
            def pt_body(p, carry):
                for q in range(C // 16):
                    sl = pl.ds(q * 16, 16)
                    acc = buf[p * K, sl]
                    for kk in range(1, K):
                        acc = jnp.maximum(acc, buf[p * K + kk, sl])
                    out_v[p, sl] = acc
                return carry

            lax.fori_loop(0, cp, pt_body, 0)
            cbase = pl.multiple_of(base_pt + ci * cp, cp)
            pltpu.sync_copy(out_v, out_hbm.at[pl.ds(cbase, cp)])

        fire(0, rows0, sem0)

        def pair_body(g, carry):
            c0 = 2 * g
            fire(c0 + 1, rows1, sem1)
            drain(rows0, sem0)
            compute(c0, rows0)

            @pl.when(c0 + 2 < nchunks)
            def _():
                fire(c0 + 2, rows0, sem0)

            drain(rows1, sem1)
            compute(c0 + 1, rows1)
            return carry

        lax.fori_loop(0, nchunks // 2, pair_body, 0)

    return gather_max


def _gather_max(table, idx2d, p_total):
    return _make_gather_max(p_total)(table, idx2d)


@functools.lru_cache(maxsize=None)
def _make_gather_max_split():
    p_half = R // 2
    per_w = R // NW
    cp = 32
    nchunks = per_w // cp
    nstreams = (cp * K) // 128
    nrows_i = per_w * K // 128
    mesh = plsc.VectorSubcoreMesh(core_axis_name="c", subcore_axis_name="s",
                                  num_cores=NC, num_subcores=NS)

    @functools.partial(
        pl.kernel,
        out_type=(jax.ShapeDtypeStruct((p_half, C), jnp.float32),
                  jax.ShapeDtypeStruct((p_half, C), jnp.float32)),
        mesh=mesh,
        compiler_params=pltpu.CompilerParams(use_tc_tiling_on_sc=False),
        scratch_types=[
            pltpu.VMEM((nrows_i, 128), jnp.int32),
            pltpu.VMEM((cp * K, C), jnp.float32),
            pltpu.VMEM((cp * K, C), jnp.float32),
            pltpu.VMEM((cp, C), jnp.float32),
            pltpu.SemaphoreType.DMA,
            pltpu.SemaphoreType.DMA,
        ],
    )
    def gather_max(table_hbm, idx0_hbm, idx1_hbm, out0_hbm, out1_hbm,
                   idx_v, rows0, rows1, out_v, sem0, sem1):
        wid = lax.axis_index("s") * NC + lax.axis_index("c")
        base_pt = wid * per_w
        lo = base_pt < p_half

        @pl.when(lo)
        def _():
            irow = pl.multiple_of(base_pt * K // 128, nrows_i)
            pltpu.sync_copy(idx0_hbm.at[pl.ds(irow, nrows_i)], idx_v)

        @pl.when(jnp.logical_not(lo))
        def _():
            irow = pl.multiple_of((base_pt - p_half) * K // 128, nrows_i)
            pltpu.sync_copy(idx1_hbm.at[pl.ds(irow, nrows_i)], idx_v)

        def fire(ci, buf, sem):
            for j in range(nstreams):
                pltpu.async_copy(table_hbm.at[idx_v.at[ci * nstreams + j]],
                                 buf.at[pl.ds(j * 128, 128)], sem)

        def drain(buf, sem):
            pltpu.make_async_copy(table_hbm.at[pl.ds(0, cp * K)],
                                  buf, sem).wait()

        def compute(ci, buf):
            def pt_body(p, carry):
                for q in range(C // 16):
                    sl = pl.ds(q * 16, 16)
                    acc = buf[p * K, sl]
                    for kk in range(1, K):
                        acc = jnp.maximum(acc, buf[p * K + kk, sl])
                    out_v[p, sl] = acc
                return carry

            lax.fori_loop(0, cp, pt_body, 0)
            cbase = pl.multiple_of(base_pt + ci * cp, cp)

            @pl.when(lo)
            def _():
                pltpu.sync_copy(out_v, out0_hbm.at[pl.ds(cbase, cp)])

            @pl.when(jnp.logical_not(lo))
            def _():
                cb = pl.multiple_of(cbase - p_half, cp)
                pltpu.sync_copy(out_v, out1_hbm.at[pl.ds(cb, cp)])

        fire(0, rows0, sem0)

        def pair_body(g, carry):
            c0 = 2 * g
            fire(c0 + 1, rows1, sem1)
            drain(rows0, sem0)
            compute(c0, rows0)

            @pl.when(c0 + 2 < nchunks)
            def _():
                fire(c0 + 2, rows0, sem0)

            drain(rows1, sem1)
            compute(c0 + 1, rows1)
            return carry

        lax.fori_loop(0, nchunks // 2, pair_body, 0)

    return gather_max


def _post_t2_body(bs0_ref, m0_ref, bs1_ref, m1_ref, f1t_ref, f2t_ref):
    f1t_ref[0] = _leaky(bs0_ref[0, 0] + m0_ref[...]).T
    f2t_ref[0] = _leaky(bs1_ref[0, 0] + m1_ref[...]).T


def _post_t2(bse, m_lo, m_hi):
    grid = (B, N // PB)
    spec_m = pl.BlockSpec((PB, C), lambda b, p: (b * (N // PB) + p, 0))
    spec_t = pl.BlockSpec((1, C, PB), lambda b, p: (b, 0, p))
    out_sh = jax.ShapeDtypeStruct((B, C, N), jnp.float32)
    return pl.pallas_call(
        _post_t2_body,
        grid=grid,
        in_specs=[
            pl.BlockSpec((1, 1, PB, C), lambda b, p: (0, b, p, 0)),
            spec_m,
            pl.BlockSpec((1, 1, PB, C), lambda b, p: (1, b, p, 0)),
            spec_m,
        ],
        out_specs=[spec_t, spec_t],
        out_shape=[out_sh, out_sh],
    )(bse, m_lo, bse, m_hi)


def _post_t0_body(bse_ref, m_ref, ft_ref):
    ft_ref[0] = _leaky(bse_ref[0, 0] + m_ref[...]).T


def _post_t0(bse, m):
    grid = (B, N // PB)
    return pl.pallas_call(
        _post_t0_body,
        grid=grid,
        in_specs=[
            pl.BlockSpec((1, 1, PB, C), lambda b, p: (0, b, p, 0)),
            pl.BlockSpec((PB, C), lambda b, p: (b * (N // PB) + p, 0)),
        ],
        out_specs=pl.BlockSpec((1, C, PB), lambda b, p: (b, 0, p)),
        out_shape=jax.ShapeDtypeStruct((B, C, N), jnp.float32),
    )(bse, m)


def kernel(pc1, pc2, feat1, feat2,
           pos1_0_w, pos1_0_b, c11_0_w, c11_0_b, c12_0_w, c12_0_b, b1_0,
           pos1_1_w, pos1_1_b, c11_1_w, c11_1_b, c12_1_w, c12_1_b, b1_1,
           pos2_0_w, pos2_0_b, c21_0_w, c21_0_b, c22_0_w, c22_0_b, b2_0):
    xcm = jnp.stack([pc1, pc2])
    xpm = xcm.transpose(0, 1, 3, 2)
    f0 = jnp.stack([feat1.transpose(0, 2, 1),
                    feat2.transpose(0, 2, 1)])

    def layer_weights(pw, pb, w1, bb1, w2, bb2, bias):
        cv1 = (bb1 + pb + bias[0, :, 0, 0]).reshape(1, C)
        cv2 = bb2.reshape(1, C)
        return w1.T, w2.T, pw.T, cv1, cv2

    wl0 = layer_weights(pos1_0_w, pos1_0_b, c11_0_w, c11_0_b,
                        c12_0_w, c12_0_b, b1_0)
    wl1 = layer_weights(pos1_1_w, pos1_1_b, c11_1_w, c11_1_b,
                        c12_1_w, c12_1_b, b1_1)
    wl2 = layer_weights(pos2_0_w, pos2_0_b, c21_0_w, c21_0_b,
                        c22_0_w, c22_0_b, b2_0)

    idx_d0 = _topk_dir(xcm, 0)
    a0, bse0 = _prep(f0, xpm, *wl0)
    m0_d0 = _gather_max(a0, idx_d0, R // 2)
    idx_d1 = _topk_dir(xcm, 1)
    m0_d1 = _gather_max(a0, idx_d1, R // 2)

    a1, bse1 = _prep_fused(bse0, m0_d0, m0_d1, xpm, *wl1)
    m1_lo, m1_hi = _make_gather_max_split()(a1, idx_d0, idx_d1)

    a2, bse2 = _prep_fused(bse1, m1_lo, m1_hi, xpm, *wl2)
    m2 = _gather_max(a2, idx_d0, R // 2)

    f1t, f2t = _post_t2(bse1, m1_lo, m1_hi)
    final = _post_t0(bse2, m2)

    return (f1t, f2t, final)


# speedup vs baseline: 43.3405x; 1.5804x over previous
import functools

import jax
import jax.numpy as jnp
from jax import lax
from jax.experimental import pallas as pl
from jax.experimental.pallas import tpu as pltpu
from jax.experimental.pallas import tpu_sc as plsc

B = 2
N = 4096
C = 64
K = 16
NDIR = 2
R = NDIR * B * N

RB = 256
PB = 1024

NC, NS = 2, 16
NW = NC * NS


def _topk_dir_body(d_idx, xs_ref, xd_ref, out_ref):
    b_idx = pl.program_id(0)
    xs = xs_ref[0, 0]
    xd = xd_ref[0, 0]
    dot = lax.dot_general(xs, xd, (((0,), (0,)), ((), ())),
                          preferred_element_type=jnp.float32)
    ns = jnp.sum(xs * xs, axis=0)[:, None]
    nd = jnp.sum(xd * xd, axis=0)[None, :]
    d = ns + nd - 2.0 * dot

    p = jnp.minimum(d[:, : N // 2], d[:, N // 2:])
    fiota = lax.broadcasted_iota(jnp.int32, (RB, N // 2), 1).astype(
        jnp.float32)
    cols = []
    for _ in range(K):
        m = jnp.min(p, axis=1, keepdims=True)
        eq = p == m
        cand = jnp.where(eq, fiota, jnp.float32(1e9))
        amin = jnp.min(cand, axis=1, keepdims=True)
        cols.append(amin)
        p = jnp.where(eq, jnp.float32(jnp.inf), p)
    out_ref[...] = jnp.concatenate(cols, axis=1).astype(jnp.int32)


def _topk_dir(xcm, d_idx):
    grid = (B, N // RB)
    return pl.pallas_call(
        functools.partial(_topk_dir_body, d_idx),
        grid=grid,
        in_specs=[
            pl.BlockSpec((1, 1, 3, RB), lambda b, r: (d_idx, b, 0, r)),
            pl.BlockSpec((1, 1, 3, N), lambda b, r: (1 - d_idx, b, 0, 0)),
        ],
        out_specs=pl.BlockSpec((RB, K), lambda b, r: (b * (N // RB) + r, 0)),
        out_shape=jax.ShapeDtypeStruct((B * N, K), jnp.int32),
    )(xcm, xcm)


@functools.lru_cache(maxsize=None)
def _make_refine(d_idx):
    per_w = B * N // NW
    tiles_per_b = (N // per_w)
    mesh = plsc.VectorSubcoreMesh(core_axis_name="c", subcore_axis_name="s",
                                  num_cores=NC, num_subcores=NS)

    @functools.partial(
        pl.kernel,
        out_type=jax.ShapeDtypeStruct((B * N, K), jnp.int32),
        mesh=mesh,
        compiler_params=pltpu.CompilerParams(use_tc_tiling_on_sc=False,
                                             needs_layout_passes=False),
        scratch_types=[
            pltpu.VMEM((N,), jnp.float32),
            pltpu.VMEM((N,), jnp.float32),
            pltpu.VMEM((N,), jnp.float32),
            pltpu.VMEM((per_w,), jnp.float32),
            pltpu.VMEM((per_w,), jnp.float32),
            pltpu.VMEM((per_w,), jnp.float32),
            pltpu.VMEM((per_w, K), jnp.int32),
            pltpu.VMEM((per_w, K), jnp.int32),
        ],
    )
    def refine(xtx_hbm, xty_hbm, xtz_hbm, xqx_hbm, xqy_hbm, xqz_hbm,
               pidx_hbm, out_hbm,
               xtx_v, xty_v, xtz_v, xqx_v, xqy_v, xqz_v, pidx_v, out_v):
        wid = lax.axis_index("s") * NC + lax.axis_index("c")
        base = pl.multiple_of(wid * per_w, per_w)
        b_off = pl.multiple_of((wid // tiles_per_b) * N, N)
        pltpu.sync_copy(xtx_hbm.at[pl.ds(b_off, N)], xtx_v)
        pltpu.sync_copy(xty_hbm.at[pl.ds(b_off, N)], xty_v)
        pltpu.sync_copy(xtz_hbm.at[pl.ds(b_off, N)], xtz_v)
        pltpu.sync_copy(xqx_hbm.at[pl.ds(base, per_w)], xqx_v)
        pltpu.sync_copy(xqy_hbm.at[pl.ds(base, per_w)], xqy_v)
        pltpu.sync_copy(xqz_hbm.at[pl.ds(base, per_w)], xqz_v)
        pltpu.sync_copy(pidx_hbm.at[pl.ds(base, per_w)], pidx_v)

        goff = d_idx * B * N + (wid // tiles_per_b) * N

        def row_body(r, carry):
            pi = pidx_v[r]
            rsplat = jnp.zeros((K,), jnp.int32) + r
            qx = plsc.load_gather(xqx_v, [rsplat])
            qy = plsc.load_gather(xqy_v, [rsplat])
            qz = plsc.load_gather(xqz_v, [rsplat])
            pj = pi + (N // 2)
            ax = plsc.load_gather(xtx_v, [pi]) - qx
            ay = plsc.load_gather(xty_v, [pi]) - qy
            az = plsc.load_gather(xtz_v, [pi]) - qz
            bx = plsc.load_gather(xtx_v, [pj]) - qx
            by = plsc.load_gather(xty_v, [pj]) - qy
            bz = plsc.load_gather(xtz_v, [pj]) - qz
            dl = ax * ax + ay * ay + az * az
            dr = bx * bx + by * by + bz * bz
            sld, sli = plsc.sort_key_val(dl, pi + goff)
            srd, sri = plsc.sort_key_val(dr, pj + goff)
            rrd = lax.rev(srd, (0,))
            rri = lax.rev(sri, (0,))
            out_v[r] = jnp.where(sld <= rrd, sli, rri)
            return carry

        lax.fori_loop(0, per_w, row_body, 0)
        pltpu.sync_copy(out_v, out_hbm.at[pl.ds(base, per_w)])

    return refine


def _flat_a(d, b, p):
    return ((1 - d) * B + b) * (N // PB) + p


def _flat_b(d, b, p):
    return (d * B + b) * (N // PB) + p


def _prep_tail(fa, fb, xa_ref, xb_ref, w1t_ref, w2t_ref, pwt_ref,
               cv1_ref, cv2_ref, a_ref, bse_ref):
    a = (jnp.dot(fa, w2t_ref[...], preferred_element_type=jnp.float32)
         + jnp.dot(xa_ref[0, 0], pwt_ref[...],
                   preferred_element_type=jnp.float32)
         + cv2_ref[...])
    bse = (jnp.dot(fb, w1t_ref[...], preferred_element_type=jnp.float32)
           - jnp.dot(xb_ref[0, 0], pwt_ref[...],
                     preferred_element_type=jnp.float32)
           + cv1_ref[...])
    a_ref[...] = a
    bse_ref[0, 0] = bse


_W_SPECS = [
    pl.BlockSpec((C, C), lambda d, b, p: (0, 0)),
    pl.BlockSpec((C, C), lambda d, b, p: (0, 0)),
    pl.BlockSpec((3, C), lambda d, b, p: (0, 0)),
    pl.BlockSpec((1, C), lambda d, b, p: (0, 0)),
    pl.BlockSpec((1, C), lambda d, b, p: (0, 0)),
]

_X_SPECS = [
    pl.BlockSpec((1, 1, PB, 3), lambda d, b, p: (1 - d, b, p, 0)),
    pl.BlockSpec((1, 1, PB, 3), lambda d, b, p: (d, b, p, 0)),
]

_OUT_SPECS = [
    pl.BlockSpec((PB, C), lambda d, b, p: (_flat_b(d, b, p), 0)),
    pl.BlockSpec((1, 1, PB, C), lambda d, b, p: (d, b, p, 0)),
]

_OUT_SHAPES = [
    jax.ShapeDtypeStruct((R, C), jnp.float32),
    jax.ShapeDtypeStruct((NDIR, B, N, C), jnp.float32),
]


def _prep_body(fa_ref, fb_ref, xa_ref, xb_ref,
               w1t_ref, w2t_ref, pwt_ref, cv1_ref, cv2_ref,
               a_ref, bse_ref):
    _prep_tail(fa_ref[0, 0], fb_ref[0, 0], xa_ref, xb_ref,
               w1t_ref, w2t_ref, pwt_ref, cv1_ref, cv2_ref, a_ref, bse_ref)


def _prep(fpm, xpm, w1t, w2t, pwt, cv1, cv2):
    grid = (NDIR, B, N // PB)
    return pl.pallas_call(
        _prep_body,
        grid=grid,
        in_specs=[
            pl.BlockSpec((1, 1, PB, C), lambda d, b, p: (1 - d, b, p, 0)),
            pl.BlockSpec((1, 1, PB, C), lambda d, b, p: (d, b, p, 0)),
            *_X_SPECS,
            *_W_SPECS,
        ],
        out_specs=_OUT_SPECS,
        out_shape=_OUT_SHAPES,
    )(fpm, fpm, xpm, xpm, w1t, w2t, pwt, cv1, cv2)


def _leaky(x):
    return jnp.where(x >= 0, x, 0.1 * x)


def _prep_fused_body(bsa_ref, bsb_ref, mlo_ref, mhi_ref, xa_ref, xb_ref,
                     w1t_ref, w2t_ref, pwt_ref, cv1_ref, cv2_ref,
                     a_ref, bse_ref):
    d0 = pl.program_id(0) == 0
    mlo = mlo_ref[...]
    mhi = mhi_ref[...]
    ma = jnp.where(d0, mhi, mlo)
    mb = jnp.where(d0, mlo, mhi)
    fa = _leaky(bsa_ref[0, 0] + ma)
    fb = _leaky(bsb_ref[0, 0] + mb)
    _prep_tail(fa, fb, xa_ref, xb_ref,
               w1t_ref, w2t_ref, pwt_ref, cv1_ref, cv2_ref, a_ref, bse_ref)


def _prep_fused(bse_prev, m_lo, m_hi, xpm, w1t, w2t, pwt, cv1, cv2):
    grid = (NDIR, B, N // PB)
    spec_m = pl.BlockSpec((PB, C), lambda d, b, p: (b * (N // PB) + p, 0))
    return pl.pallas_call(
        _prep_fused_body,
        grid=grid,
        in_specs=[
            pl.BlockSpec((1, 1, PB, C), lambda d, b, p: (1 - d, b, p, 0)),
            pl.BlockSpec((1, 1, PB, C), lambda d, b, p: (d, b, p, 0)),
            spec_m, spec_m,
            *_X_SPECS,
            *_W_SPECS,
        ],
        out_specs=_OUT_SPECS,
        out_shape=_OUT_SHAPES,
    )(bse_prev, bse_prev, m_lo, m_hi, xpm, xpm,
      w1t, w2t, pwt, cv1, cv2)


@functools.lru_cache(maxsize=None)
def _make_gather_max(p_total):
    per_w = p_total // NW
    cp = 32
    nchunks = per_w // cp
    nstreams = (cp * K) // 128
    nrows_i = per_w * K // 128
    mesh = plsc.VectorSubcoreMesh(core_axis_name="c", subcore_axis_name="s",
                                  num_cores=NC, num_subcores=NS)

    @functools.partial(
        pl.kernel,
        out_type=jax.ShapeDtypeStruct((p_total, C), jnp.float32),
        mesh=mesh,
        compiler_params=pltpu.CompilerParams(use_tc_tiling_on_sc=False),
        scratch_types=[
            pltpu.VMEM((nrows_i, 128), jnp.int32),
            pltpu.VMEM((cp * K, C), jnp.float32),
            pltpu.VMEM((cp * K, C), jnp.float32),
            pltpu.VMEM((cp, C), jnp.float32),
            pltpu.SemaphoreType.DMA,
            pltpu.SemaphoreType.DMA,
        ],
    )
    def gather_max(table_hbm, idx_hbm, out_hbm,
                   idx_v, rows0, rows1, out_v, sem0, sem1):
        wid = lax.axis_index("s") * NC + lax.axis_index("c")
        base_pt = wid * per_w
        irow = pl.multiple_of(base_pt * K // 128, nrows_i)
        pltpu.sync_copy(idx_hbm.at[pl.ds(irow, nrows_i)], idx_v)

        def fire(ci, buf, sem):
            for j in range(nstreams):
                pltpu.async_copy(table_hbm.at[idx_v.at[ci * nstreams + j]],
                                 buf.at[pl.ds(j * 128, 128)], sem)

        def drain(buf, sem):
            pltpu.make_async_copy(table_hbm.at[pl.ds(0, cp * K)],
                                  buf, sem).wait()

        def compute(ci, buf):
            def pt_body(p, carry):
                for q in range(C // 16):
                    sl = pl.ds(q * 16, 16)
                    acc = buf[p * K, sl]
                    for kk in range(1, K):
                        acc = jnp.maximum(acc, buf[p * K + kk, sl])
                    out_v[p, sl] = acc
                return carry

            lax.fori_loop(0, cp, pt_body, 0)
            cbase = pl.multiple_of(base_pt + ci * cp, cp)
            pltpu.sync_copy(out_v, out_hbm.at[pl.ds(cbase, cp)])

        fire(0, rows0, sem0)

        def pair_body(g, carry):
            c0 = 2 * g
            fire(c0 + 1, rows1, sem1)
            drain(rows0, sem0)
            compute(c0, rows0)

            @pl.when(c0 + 2 < nchunks)
            def _():
                fire(c0 + 2, rows0, sem0)

            drain(rows1, sem1)
            compute(c0 + 1, rows1)
            return carry

        lax.fori_loop(0, nchunks // 2, pair_body, 0)

    return gather_max


def _gather_max(table, idx2d, p_total):
    return _make_gather_max(p_total)(table, idx2d)


@functools.lru_cache(maxsize=None)
def _make_gather_max_split():
    p_half = R // 2
    per_w = R // NW
    cp = 32
    nchunks = per_w // cp
    nstreams = (cp * K) // 128
    nrows_i = per_w * K // 128
    mesh = plsc.VectorSubcoreMesh(core_axis_name="c", subcore_axis_name="s",
                                  num_cores=NC, num_subcores=NS)

    @functools.partial(
        pl.kernel,
        out_type=(jax.ShapeDtypeStruct((p_half, C), jnp.float32),
                  jax.ShapeDtypeStruct((p_half, C), jnp.float32)),
        mesh=mesh,
        compiler_params=pltpu.CompilerParams(use_tc_tiling_on_sc=False),
        scratch_types=[
            pltpu.VMEM((nrows_i, 128), jnp.int32),
            pltpu.VMEM((cp * K, C), jnp.float32),
            pltpu.VMEM((cp * K, C), jnp.float32),
            pltpu.VMEM((cp, C), jnp.float32),
            pltpu.SemaphoreType.DMA,
            pltpu.SemaphoreType.DMA,
        ],
    )
    def gather_max(table_hbm, idx0_hbm, idx1_hbm, out0_hbm, out1_hbm,
                   idx_v, rows0, rows1, out_v, sem0, sem1):
        wid = lax.axis_index("s") * NC + lax.axis_index("c")
        base_pt = wid * per_w
        lo = base_pt < p_half

        @pl.when(lo)
        def _():
            irow = pl.multiple_of(base_pt * K // 128, nrows_i)
            pltpu.sync_copy(idx0_hbm.at[pl.ds(irow, nrows_i)], idx_v)

        @pl.when(jnp.logical_not(lo))
        def _():
            irow = pl.multiple_of((base_pt - p_half) * K // 128, nrows_i)
            pltpu.sync_copy(idx1_hbm.at[pl.ds(irow, nrows_i)], idx_v)

        def fire(ci, buf, sem):
            for j in range(nstreams):
                pltpu.async_copy(table_hbm.at[idx_v.at[ci * nstreams + j]],
                                 buf.at[pl.ds(j * 128, 128)], sem)

        def drain(buf, sem):
            pltpu.make_async_copy(table_hbm.at[pl.ds(0, cp * K)],
                                  buf, sem).wait()

        def compute(ci, buf):
            def pt_body(p, carry):
                for q in range(C // 16):
                    sl = pl.ds(q * 16, 16)
                    acc = buf[p * K, sl]
                    for kk in range(1, K):
                        acc = jnp.maximum(acc, buf[p * K + kk, sl])
                    out_v[p, sl] = acc
                return carry

            lax.fori_loop(0, cp, pt_body, 0)
            cbase = pl.multiple_of(base_pt + ci * cp, cp)

            @pl.when(lo)
            def _():
                pltpu.sync_copy(out_v, out0_hbm.at[pl.ds(cbase, cp)])

            @pl.when(jnp.logical_not(lo))
            def _():
                cb = pl.multiple_of(cbase - p_half, cp)
                pltpu.sync_copy(out_v, out1_hbm.at[pl.ds(cb, cp)])

        fire(0, rows0, sem0)

        def pair_body(g, carry):
            c0 = 2 * g
            fire(c0 + 1, rows1, sem1)
            drain(rows0, sem0)
            compute(c0, rows0)

            @pl.when(c0 + 2 < nchunks)
            def _():
                fire(c0 + 2, rows0, sem0)

            drain(rows1, sem1)
            compute(c0 + 1, rows1)
            return carry

        lax.fori_loop(0, nchunks // 2, pair_body, 0)

    return gather_max


def _post_t2_body(bs0_ref, m0_ref, bs1_ref, m1_ref, f1t_ref, f2t_ref):
    f1t_ref[0] = _leaky(bs0_ref[0, 0] + m0_ref[...]).T
    f2t_ref[0] = _leaky(bs1_ref[0, 0] + m1_ref[...]).T


def _post_t2(bse, m_lo, m_hi):
    grid = (B, N // PB)
    spec_m = pl.BlockSpec((PB, C), lambda b, p: (b * (N // PB) + p, 0))
    spec_t = pl.BlockSpec((1, C, PB), lambda b, p: (b, 0, p))
    out_sh = jax.ShapeDtypeStruct((B, C, N), jnp.float32)
    return pl.pallas_call(
        _post_t2_body,
        grid=grid,
        in_specs=[
            pl.BlockSpec((1, 1, PB, C), lambda b, p: (0, b, p, 0)),
            spec_m,
            pl.BlockSpec((1, 1, PB, C), lambda b, p: (1, b, p, 0)),
            spec_m,
        ],
        out_specs=[spec_t, spec_t],
        out_shape=[out_sh, out_sh],
    )(bse, m_lo, bse, m_hi)


def _post_t0_body(bse_ref, m_ref, ft_ref):
    ft_ref[0] = _leaky(bse_ref[0, 0] + m_ref[...]).T


def _post_t0(bse, m):
    grid = (B, N // PB)
    return pl.pallas_call(
        _post_t0_body,
        grid=grid,
        in_specs=[
            pl.BlockSpec((1, 1, PB, C), lambda b, p: (0, b, p, 0)),
            pl.BlockSpec((PB, C), lambda b, p: (b * (N // PB) + p, 0)),
        ],
        out_specs=pl.BlockSpec((1, C, PB), lambda b, p: (b, 0, p)),
        out_shape=jax.ShapeDtypeStruct((B, C, N), jnp.float32),
    )(bse, m)


def kernel(pc1, pc2, feat1, feat2,
           pos1_0_w, pos1_0_b, c11_0_w, c11_0_b, c12_0_w, c12_0_b, b1_0,
           pos1_1_w, pos1_1_b, c11_1_w, c11_1_b, c12_1_w, c12_1_b, b1_1,
           pos2_0_w, pos2_0_b, c21_0_w, c21_0_b, c22_0_w, c22_0_b, b2_0):
    xcm = jnp.stack([pc1, pc2])
    xpm = xcm.transpose(0, 1, 3, 2)
    f0 = jnp.stack([feat1.transpose(0, 2, 1),
                    feat2.transpose(0, 2, 1)])

    def layer_weights(pw, pb, w1, bb1, w2, bb2, bias):
        cv1 = (bb1 + pb + bias[0, :, 0, 0]).reshape(1, C)
        cv2 = bb2.reshape(1, C)
        return w1.T, w2.T, pw.T, cv1, cv2

    wl0 = layer_weights(pos1_0_w, pos1_0_b, c11_0_w, c11_0_b,
                        c12_0_w, c12_0_b, b1_0)
    wl1 = layer_weights(pos1_1_w, pos1_1_b, c11_1_w, c11_1_b,
                        c12_1_w, c12_1_b, b1_1)
    wl2 = layer_weights(pos2_0_w, pos2_0_b, c21_0_w, c21_0_b,
                        c22_0_w, c22_0_b, b2_0)

    xc = [[xcm[p, :, c, :].reshape(B * N) for c in range(3)]
          for p in range(2)]

    pidx_d0 = _topk_dir(xcm, 0)
    idx_d0 = _make_refine(0)(
        *xc[1], *xc[0], pidx_d0).reshape(B * N * K // 128, 128)
    a0, bse0 = _prep(f0, xpm, *wl0)
    m0_d0 = _gather_max(a0, idx_d0, R // 2)
    pidx_d1 = _topk_dir(xcm, 1)
    idx_d1 = _make_refine(1)(
        *xc[0], *xc[1], pidx_d1).reshape(B * N * K // 128, 128)
    m0_d1 = _gather_max(a0, idx_d1, R // 2)

    a1, bse1 = _prep_fused(bse0, m0_d0, m0_d1, xpm, *wl1)
    m1_lo, m1_hi = _make_gather_max_split()(a1, idx_d0, idx_d1)

    a2, bse2 = _prep_fused(bse1, m1_lo, m1_hi, xpm, *wl2)
    m2 = _gather_max(a2, idx_d0, R // 2)

    f1t, f2t = _post_t2(bse1, m1_lo, m1_hi)
    final = _post_t0(bse2, m2)

    return (f1t, f2t, final)
